# Initial kernel scaffold; baseline (speedup 1.0000x reference)
import jax
import jax.numpy as jnp
from jax.experimental import pallas as pl


def kernel(attack_tensor, non_attack_tensor):
    raise NotImplementedError("write your pallas kernel here")


# speedup vs baseline: 3.9351x; 3.9351x over previous
import functools

import jax
import jax.numpy as jnp
from jax import lax
from jax.experimental import pallas as pl
from jax.experimental.pallas import tpu as pltpu
from jax.experimental.pallas import tpu_sc as plsc

_ROWS = 128
_COLS = 32768
_TOPK = 64
_NCORES = 2
_NSUB = 16
_NWORK = _NCORES * _NSUB
_ROWS_PER = _ROWS // _NWORK
_L = 16
_NBINS = 256
_MSB = jnp.int32(-2147483648)


def _zero_hist(hist_v):
    z = jnp.zeros((_L,), jnp.int32)

    def zb(i, _):
        hist_v[pl.ds(i * _L, _L)] = z
        return 0

    lax.fori_loop(0, (_NBINS * _L) // _L, zb, 0)


def _scan_hist(hist_v, histr_v, krem):
    def lr(c, _):
        acc = hist_v[pl.ds(c * _L, _L)]
        for lane in range(1, _L):
            acc = acc + hist_v[pl.ds(lane * _NBINS + c * _L, _L)]
        histr_v[pl.ds(c * _L, _L)] = acc
        return 0

    lax.fori_loop(0, _NBINS // _L, lr, 0)

    def sb(j, carry):
        cum, bstar, above = carry
        c = _NBINS // _L - 1 - j
        v = histr_v[pl.ds(c * _L, _L)]
        for lane in range(_L - 1, -1, -1):
            cnt = v[lane]
            d = c * _L + lane
            hit = jnp.logical_and(cum < krem, cum + cnt >= krem)
            bstar = jnp.where(hit, d, bstar)
            above = jnp.where(hit, cum, above)
            cum = cum + cnt
        return (cum, bstar, above)

    _, bstar, above = lax.fori_loop(
        0, _NBINS // _L, sb, (jnp.int32(0), jnp.int32(0), jnp.int32(0))
    )
    return bstar, krem - above


def _compact(src, dst, n16, shift, bstar):
    def cb(i, off):
        u = src[pl.ds(i * _L, _L)]
        digit = lax.shift_right_logical(u, shift) & 0xFF
        keep = digit == bstar
        plsc.store_compressed(dst.at[pl.ds(off, _L)], u, mask=keep)
        return off + jnp.sum(keep.astype(jnp.int32))

    off = lax.fori_loop(0, n16, cb, jnp.int32(0))
    dst[pl.ds(off, _L)] = jnp.zeros((_L,), jnp.int32)
    return (off + _L - 1) // _L


def _sc_body(attack_hbm, out_hbm, row_v, cand_a, cand_b, hist_v, histr_v,
             tf_v):
    wid = lax.axis_index("s") * _NCORES + lax.axis_index("c")
    lane_off = lax.iota(jnp.int32, _L) * _NBINS
    ones = jnp.ones((_L,), jnp.int32)

    def row_body(r, _):
        row = wid * _ROWS_PER + r
        pltpu.sync_copy(attack_hbm.at[row], row_v)

        _zero_hist(hist_v)

        def hb1(i, _):
            ib = row_v[pl.ds(i * _L, _L)]
            u = ib ^ ((ib >> 31) | _MSB)
            cand_a[pl.ds(i * _L, _L)] = u
            digit = lax.shift_right_logical(u, 24)
            plsc.addupdate_scatter(hist_v, [lane_off + digit], ones)
            return 0

        lax.fori_loop(0, _COLS // _L, hb1, 0)
        b1, krem = _scan_hist(hist_v, histr_v, jnp.int32(_TOPK))
        n16 = _compact(cand_a, cand_b, jnp.int32(_COLS // _L), 24, b1)

        def hist_round(src, n16, shift, krem):
            _zero_hist(hist_v)

            def hb(i, _):
                u = src[pl.ds(i * _L, _L)]
                digit = lax.shift_right_logical(u, shift) & 0xFF
                plsc.addupdate_scatter(hist_v, [lane_off + digit], ones)
                return 0

            lax.fori_loop(0, n16, hb, 0)
            return _scan_hist(hist_v, histr_v, krem)

        b2, krem = hist_round(cand_b, n16, 16, krem)
        n16 = _compact(cand_b, cand_a, n16, 16, b2)
        b3, krem = hist_round(cand_a, n16, 8, krem)
        n16 = _compact(cand_a, cand_b, n16, 8, b3)
        b4, _ = hist_round(cand_b, n16, 0, krem)

        tu = (b1 << 24) | (b2 << 16) | (b3 << 8) | b4
        tf_v[...] = jnp.full((_L,), tu, jnp.int32)
        pltpu.sync_copy(tf_v, out_hbm.at[row])
        return 0

    lax.fori_loop(0, _ROWS_PER, row_body, 0)


_sc_thresholds = functools.partial(
    pl.kernel,
    out_type=jax.ShapeDtypeStruct((_ROWS, _L), jnp.int32),
    mesh=plsc.VectorSubcoreMesh(
        core_axis_name="c", subcore_axis_name="s",
        num_cores=_NCORES, num_subcores=_NSUB,
    ),
    compiler_params=pltpu.CompilerParams(needs_layout_passes=False),
    scratch_types=[
        pltpu.VMEM((_COLS,), jnp.int32),
        pltpu.VMEM((_COLS + _L,), jnp.int32),
        pltpu.VMEM((_COLS + _L,), jnp.int32),
        pltpu.VMEM((_NBINS * _L,), jnp.int32),
        pltpu.VMEM((_NBINS,), jnp.int32),
        pltpu.VMEM((_L,), jnp.int32),
    ],
)(_sc_body)


def _loss_body(a_ref, n_ref, t_ref, o_ref):
    a = a_ref[...]
    na = n_ref[...]
    t = t_ref[...]
    keep = a < t
    target = jnp.where(keep, a, 0.0)
    m = jnp.max(target)
    scale = target / ((m + 0.1) * 0.5)
    loss1 = -jnp.sum(jnp.log(1.0 - 0.5 * scale))
    tnon = jnp.where(keep, na, 0.0)
    c = jnp.min(a - na, axis=0, keepdims=True)
    z = 10.0 * (target - tnon - c)
    loss2 = jnp.sum(1.0 / (1.0 + jnp.exp(-z)))
    o_ref[...] = jnp.broadcast_to(loss1 + loss2, (1, 1))


def kernel(attack_tensor, non_attack_tensor):
    a_bits = lax.bitcast_convert_type(attack_tensor, jnp.int32)
    tu16 = _sc_thresholds(a_bits)
    tu = tu16[:, :1]
    orig = jnp.where(tu < 0, tu ^ _MSB, ~tu)
    t = lax.bitcast_convert_type(orig, jnp.float32)
    a_tl = attack_tensor[:, ::1024]
    na_tl = non_attack_tensor[:, ::1024]
    out = pl.pallas_call(
        _loss_body,
        out_shape=jax.ShapeDtypeStruct((1, 1), jnp.float32),
    )(a_tl, na_tl, t)
    return out[0, 0]


# speedup vs baseline: 3.9386x; 1.0009x over previous
import functools

import jax
import jax.numpy as jnp
from jax import lax
from jax.experimental import pallas as pl
from jax.experimental.pallas import tpu as pltpu
from jax.experimental.pallas import tpu_sc as plsc

_ROWS = 128
_COLS = 32768
_TOPK = 64
_NCORES = 2
_NSUB = 16
_NWORK = _NCORES * _NSUB
_ROWS_PER = _ROWS // _NWORK
_L = 16
_NBINS = 256
_UNROLL = 4
_MSB = jnp.int32(-2147483648)


def _zero_hist(hist_v):
    z = jnp.zeros((_L,), jnp.int32)

    def zb(i, _):
        for k in range(_UNROLL):
            hist_v[pl.ds(i * _L * _UNROLL + k * _L, _L)] = z
        return 0

    lax.fori_loop(0, (_NBINS * _L) // (_L * _UNROLL), zb, 0)


def _scan_hist(hist_v, histr_v, krem):
    def lr(c, _):
        acc = hist_v[pl.ds(c * _L, _L)]
        for lane in range(1, _L):
            acc = acc + hist_v[pl.ds(lane * _NBINS + c * _L, _L)]
        histr_v[pl.ds(c * _L, _L)] = acc
        return 0

    lax.fori_loop(0, _NBINS // _L, lr, 0)

    def sb(j, carry):
        cum, bstar, above = carry
        c = _NBINS // _L - 1 - j
        v = histr_v[pl.ds(c * _L, _L)]
        for lane in range(_L - 1, -1, -1):
            cnt = v[lane]
            d = c * _L + lane
            hit = jnp.logical_and(cum < krem, cum + cnt >= krem)
            bstar = jnp.where(hit, d, bstar)
            above = jnp.where(hit, cum, above)
            cum = cum + cnt
        return (cum, bstar, above)

    _, bstar, above = lax.fori_loop(
        0, _NBINS // _L, sb, (jnp.int32(0), jnp.int32(0), jnp.int32(0))
    )
    return bstar, krem - above


def _sc_body(attack_hbm, out_hbm, row_a, row_b, cand, hist_v, histr_v,
             tf_v, sem):
    wid = lax.axis_index("s") * _NCORES + lax.axis_index("c")
    lane_off = lax.iota(jnp.int32, _L) * _NBINS
    ones = jnp.ones((_L,), jnp.int32)
    zeros = jnp.zeros((_L,), jnp.int32)
    row0 = wid * _ROWS_PER

    def key16(src, i):
        ib = src[pl.ds(i * _L, _L)]
        return ib ^ ((ib >> 31) | _MSB)

    def compact_inplace(n16, shift, bstar):
        def cb(i, off):
            u = cand[pl.ds(i * _L, _L)]
            keep = (lax.shift_right_logical(u, shift) & 0xFF) == bstar
            plsc.store_compressed(cand.at[pl.ds(off, _L)], u, mask=keep)
            return off + plsc.all_reduce_population_count(keep)[0]

        off = lax.fori_loop(0, n16, cb, jnp.int32(0))
        cand[pl.ds(off, _L)] = zeros
        return (off + _L - 1) // _L

    def hist_round(n16, shift, krem):
        _zero_hist(hist_v)

        def hb(i, _):
            u = cand[pl.ds(i * _L, _L)]
            digit = lax.shift_right_logical(u, shift) & 0xFF
            plsc.addupdate_scatter(hist_v, [lane_off + digit], ones)
            return 0

        lax.fori_loop(0, n16, hb, 0)
        return _scan_hist(hist_v, histr_v, krem)

    def process_row(row, src):
        _zero_hist(hist_v)

        def hb1(i, _):
            for k in range(_UNROLL):
                u = key16(src, i * _UNROLL + k)
                digit = lax.shift_right_logical(u, 24)
                plsc.addupdate_scatter(hist_v, [lane_off + digit], ones)
            return 0

        lax.fori_loop(0, _COLS // (_L * _UNROLL), hb1, 0)
        b1, krem = _scan_hist(hist_v, histr_v, jnp.int32(_TOPK))

        def cb1(i, off):
            for k in range(_UNROLL):
                u = key16(src, i * _UNROLL + k)
                keep = lax.shift_right_logical(u, 24) == b1
                plsc.store_compressed(cand.at[pl.ds(off, _L)], u, mask=keep)
                off = off + plsc.all_reduce_population_count(keep)[0]
            return off

        off = lax.fori_loop(0, _COLS // (_L * _UNROLL), cb1, jnp.int32(0))
        cand[pl.ds(off, _L)] = zeros
        n16 = (off + _L - 1) // _L

        b2, krem = hist_round(n16, 16, krem)
        n16 = compact_inplace(n16, 16, b2)
        b3, krem = hist_round(n16, 8, krem)
        n16 = compact_inplace(n16, 8, b3)
        b4, _ = hist_round(n16, 0, krem)

        tu = (b1 << 24) | (b2 << 16) | (b3 << 8) | b4
        tf_v[...] = jnp.full((_L,), tu, jnp.int32)
        pltpu.sync_copy(tf_v, out_hbm.at[row])

    bufs = [row_a, row_b]
    copies = [pltpu.async_copy(attack_hbm.at[row0], row_a, sem)]
    for r in range(_ROWS_PER):
        copies[r].wait()
        if r + 1 < _ROWS_PER:
            copies.append(
                pltpu.async_copy(attack_hbm.at[row0 + r + 1],
                                 bufs[(r + 1) % 2], sem)
            )
        process_row(row0 + r, bufs[r % 2])


_sc_thresholds = functools.partial(
    pl.kernel,
    out_type=jax.ShapeDtypeStruct((_ROWS, _L), jnp.int32),
    mesh=plsc.VectorSubcoreMesh(
        core_axis_name="c", subcore_axis_name="s",
        num_cores=_NCORES, num_subcores=_NSUB,
    ),
    compiler_params=pltpu.CompilerParams(needs_layout_passes=False),
    scratch_types=[
        pltpu.VMEM((_COLS,), jnp.int32),
        pltpu.VMEM((_COLS,), jnp.int32),
        pltpu.VMEM((_COLS + _L,), jnp.int32),
        pltpu.VMEM((_NBINS * _L,), jnp.int32),
        pltpu.VMEM((_NBINS,), jnp.int32),
        pltpu.VMEM((_L,), jnp.int32),
        pltpu.SemaphoreType.DMA,
    ],
)(_sc_body)


def _loss_body(a_ref, n_ref, t_ref, o_ref):
    a = a_ref[...]
    na = n_ref[...]
    t = t_ref[...]
    keep = a < t
    target = jnp.where(keep, a, 0.0)
    m = jnp.max(target)
    scale = target / ((m + 0.1) * 0.5)
    loss1 = -jnp.sum(jnp.log(1.0 - 0.5 * scale))
    tnon = jnp.where(keep, na, 0.0)
    c = jnp.min(a - na, axis=0, keepdims=True)
    z = 10.0 * (target - tnon - c)
    loss2 = jnp.sum(1.0 / (1.0 + jnp.exp(-z)))
    o_ref[...] = jnp.broadcast_to(loss1 + loss2, (1, 1))


def kernel(attack_tensor, non_attack_tensor):
    a_bits = lax.bitcast_convert_type(attack_tensor, jnp.int32)
    tu16 = _sc_thresholds(a_bits)
    tu = tu16[:, :1]
    orig = jnp.where(tu < 0, tu ^ _MSB, ~tu)
    t = lax.bitcast_convert_type(orig, jnp.float32)
    a_tl = attack_tensor[:, ::1024]
    na_tl = non_attack_tensor[:, ::1024]
    out = pl.pallas_call(
        _loss_body,
        out_shape=jax.ShapeDtypeStruct((1, 1), jnp.float32),
    )(a_tl, na_tl, t)
    return out[0, 0]
